# baseline (speedup 1.0000x reference)
import functools

import jax
import jax.numpy as jnp
from jax import lax
from jax.experimental import pallas as pl
from jax.experimental.pallas import tpu as pltpu
from jax.experimental.pallas import tpu_sc as plsc

_SHIFT = 8192
_NUM_CORES = 2
_NUM_SUBCORES = 16


def _shift_gather(wav, offs, length):
    sources, batch, channels, _ = wav.shape
    rows = sources * batch * channels
    num_workers = _NUM_CORES * _NUM_SUBCORES
    rows_per_worker = rows // num_workers

    chunk = 31744
    num_chunks = length // chunk

    mesh = plsc.VectorSubcoreMesh(
        core_axis_name="core", subcore_axis_name="subcore",
        num_cores=_NUM_CORES, num_subcores=_NUM_SUBCORES,
    )

    @functools.partial(
        pl.kernel,
        out_type=jax.ShapeDtypeStruct(
            (sources, batch, channels, length), jnp.float32
        ),
        mesh=mesh,
        scratch_types=[
            pltpu.VMEM((num_workers, 16), jnp.int32),
            pltpu.VMEM((chunk + 8,), jnp.float32),
            pltpu.VMEM((chunk + 8,), jnp.float32),
            pltpu.VMEM((chunk,), jnp.float32),
            pltpu.VMEM((chunk,), jnp.float32),
            pltpu.SemaphoreType.DMA,
            pltpu.SemaphoreType.DMA,
            pltpu.SemaphoreType.DMA,
            pltpu.SemaphoreType.DMA,
        ],
        compiler_params=pltpu.CompilerParams(use_tc_tiling_on_sc=False),
    )
    def shift_copy(wav_hbm, off_hbm, out_hbm, off_v,
                   ib0, ib1, ob0, ob1, si0, si1, so0, so1):
        ibufs, obufs = (ib0, ib1), (ob0, ob1)
        isems, osems = (si0, si1), (so0, so1)
        wid = lax.axis_index("subcore") * _NUM_CORES + lax.axis_index("core")
        pltpu.sync_copy(off_hbm, off_v)
        my_offs = off_v[wid]

        idxs = []
        bases = []
        rems = []
        for r in range(rows_per_worker):
            off = my_offs[r]
            row = wid * rows_per_worker + r
            s = row // (batch * channels)
            b = (row // channels) % batch
            c = row % channels
            idxs.append((s, b, c))
            bases.append(pl.multiple_of((off // 8) * 8, 8))
            rems.append(off - bases[-1])

        tasks = [(r, j) for r in range(rows_per_worker)
                 for j in range(num_chunks)]
        ntasks = len(tasks)

        def in_desc(tt):
            r, j = tasks[tt]
            s, b, c = idxs[r]
            return pltpu.make_async_copy(
                wav_hbm.at[s, b, c, pl.ds(bases[r] + j * chunk, chunk + 8)],
                ibufs[tt % 2], isems[tt % 2])

        def out_desc(tt):
            r, j = tasks[tt]
            s, b, c = idxs[r]
            return pltpu.make_async_copy(
                obufs[tt % 2],
                out_hbm.at[s, b, c, pl.ds(j * chunk, chunk)],
                osems[tt % 2])

        def shift(tt):
            r, _ = tasks[tt]
            rem = rems[r]
            ib, ob = ibufs[tt % 2], obufs[tt % 2]

            def body(g, carry):
                base = g * 256
                vals = [ib[pl.ds(rem + base + u * 16, 16)] for u in range(16)]
                for u, v in enumerate(vals):
                    ob[pl.ds(base + u * 16, 16)] = v
                return carry

            lax.fori_loop(0, chunk // 256, body, 0)

        in_desc(0).start()
        for tt in range(ntasks):
            if tt + 1 < ntasks:
                in_desc(tt + 1).start()
            in_desc(tt).wait()
            if tt >= 2:
                out_desc(tt - 2).wait()
            shift(tt)
            out_desc(tt).start()
        out_desc(ntasks - 2).wait()
        out_desc(ntasks - 1).wait()

    return shift_copy(wav, offs)


def kernel(wav):
    sources, batch, channels, length0 = wav.shape
    length = length0 - _SHIFT
    okey = jax.random.fold_in(jax.random.key(0), 1)
    offsets = jax.random.randint(
        okey, (sources, batch, 1, 1), 0, _SHIFT, dtype=jnp.int32
    )
    rows = sources * batch * channels
    num_workers = _NUM_CORES * _NUM_SUBCORES
    offs = jnp.broadcast_to(offsets, (sources, batch, channels, 1)).reshape(
        num_workers, rows // num_workers
    )
    offs = jnp.pad(offs, ((0, 0), (0, 16 - rows // num_workers)))
    return _shift_gather(wav, offs, length)


# speedup vs baseline: 3.7306x; 3.7306x over previous
import functools

import jax
import jax.numpy as jnp
from jax import lax
from jax.experimental import pallas as pl
from jax.experimental.pallas import tpu as pltpu
from jax.experimental.pallas import tpu_sc as plsc

_SHIFT = 8192
_NUM_CORES = 2
_NUM_SUBCORES = 16


def _shift_gather(wav, offs, length):
    sources, batch, channels, _ = wav.shape
    pairs = sources * batch
    num_workers = _NUM_CORES * _NUM_SUBCORES
    pairs_per_worker = pairs // num_workers

    chunk = 8192
    num_chunks = length // chunk

    mesh = plsc.VectorSubcoreMesh(
        core_axis_name="core", subcore_axis_name="subcore",
        num_cores=_NUM_CORES, num_subcores=_NUM_SUBCORES,
    )

    @functools.partial(
        pl.kernel,
        out_type=jax.ShapeDtypeStruct(
            (sources, batch, channels, length), jnp.float32
        ),
        mesh=mesh,
        scratch_types=[
            pltpu.VMEM((num_workers, 16), jnp.int32),
            pltpu.VMEM((channels, chunk + 128), jnp.float32),
            pltpu.VMEM((channels, chunk + 128), jnp.float32),
            pltpu.VMEM((channels, chunk), jnp.float32),
            pltpu.VMEM((channels, chunk), jnp.float32),
            pltpu.SemaphoreType.DMA,
            pltpu.SemaphoreType.DMA,
            pltpu.SemaphoreType.DMA,
            pltpu.SemaphoreType.DMA,
        ],
        compiler_params=pltpu.CompilerParams(use_tc_tiling_on_sc=True),
    )
    def shift_copy(wav_hbm, off_hbm, out_hbm, off_v,
                   ib0, ib1, ob0, ob1, si0, si1, so0, so1):
        ibufs, obufs = (ib0, ib1), (ob0, ob1)
        isems, osems = (si0, si1), (so0, so1)
        wid = lax.axis_index("subcore") * _NUM_CORES + lax.axis_index("core")
        pltpu.sync_copy(off_hbm, off_v)
        my_offs = off_v[wid]

        iota16 = lax.broadcasted_iota(jnp.int32, (16,), 0)
        idxs = []
        bases = []
        rem16s = []
        rots = []
        masks = []
        for k in range(pairs_per_worker):
            off = my_offs[k]
            p = wid * pairs_per_worker + k
            s = p // batch
            b = p % batch
            idxs.append((s, b))
            bases.append(pl.multiple_of((off // 128) * 128, 128))
            rem = off - bases[-1]
            r2 = jnp.bitwise_and(off, 15)
            rem16s.append(pl.multiple_of(rem - r2, 16))
            rots.append(jnp.bitwise_and(iota16 + r2, 15))
            masks.append(iota16 < 16 - r2)

        tasks = [(k, j) for k in range(pairs_per_worker)
                 for j in range(num_chunks)]
        ntasks = len(tasks)

        def in_desc(tt):
            k, j = tasks[tt]
            s, b = idxs[k]
            return pltpu.make_async_copy(
                wav_hbm.at[s, b, :, pl.ds(bases[k] + j * chunk, chunk + 128)],
                ibufs[tt % 2], isems[tt % 2])

        def out_desc(tt):
            k, j = tasks[tt]
            s, b = idxs[k]
            return pltpu.make_async_copy(
                obufs[tt % 2],
                out_hbm.at[s, b, :, pl.ds(j * chunk, chunk)],
                osems[tt % 2])

        def shift(tt):
            k, _ = tasks[tt]
            rem16, rot, mask = rem16s[k], rots[k], masks[k]
            ib, ob = ibufs[tt % 2], obufs[tt % 2]

            def body(g, carry):
                base = g * 256
                for c in range(channels):
                    pos = rem16 + base
                    vals = [ib[c, pl.ds(pos + u * 16, 16)]
                            for u in range(17)]
                    for u in range(16):
                        g0 = vals[u].at[rot].get(mode="promise_in_bounds")
                        g1 = vals[u + 1].at[rot].get(mode="promise_in_bounds")
                        ob[c, pl.ds(base + u * 16, 16)] = jnp.where(
                            mask, g0, g1)
                return carry

            lax.fori_loop(0, chunk // 256, body, 0)

        in_desc(0).start()
        for tt in range(ntasks):
            if tt + 1 < ntasks:
                in_desc(tt + 1).start()
            in_desc(tt).wait()
            if tt >= 2:
                out_desc(tt - 2).wait()
            shift(tt)
            out_desc(tt).start()
        out_desc(ntasks - 2).wait()
        out_desc(ntasks - 1).wait()

    return shift_copy(wav, offs)


def kernel(wav):
    sources, batch, channels, length0 = wav.shape
    length = length0 - _SHIFT
    okey = jax.random.fold_in(jax.random.key(0), 1)
    offsets = jax.random.randint(
        okey, (sources, batch, 1, 1), 0, _SHIFT, dtype=jnp.int32
    )
    pairs = sources * batch
    num_workers = _NUM_CORES * _NUM_SUBCORES
    offs = offsets.reshape(num_workers, pairs // num_workers)
    offs = jnp.pad(offs, ((0, 0), (0, 16 - pairs // num_workers)))
    return _shift_gather(wav, offs, length)


# speedup vs baseline: 4.0586x; 1.0879x over previous
import functools

import jax
import jax.numpy as jnp
from jax import lax
from jax.experimental import pallas as pl
from jax.experimental.pallas import tpu as pltpu
from jax.experimental.pallas import tpu_sc as plsc

_SHIFT = 8192
_NUM_CORES = 2
_NUM_SUBCORES = 16


def _shift_gather(wav, offs, length):
    sources, batch, channels, _ = wav.shape
    pairs = sources * batch
    num_workers = _NUM_CORES * _NUM_SUBCORES
    pairs_per_worker = pairs // num_workers

    chunk = 8192
    num_chunks = length // chunk

    mesh = plsc.VectorSubcoreMesh(
        core_axis_name="core", subcore_axis_name="subcore",
        num_cores=_NUM_CORES, num_subcores=_NUM_SUBCORES,
    )

    @functools.partial(
        pl.kernel,
        out_type=jax.ShapeDtypeStruct(
            (sources, batch, channels, length), jnp.float32
        ),
        mesh=mesh,
        scratch_types=[
            pltpu.VMEM((num_workers, 16), jnp.int32),
            pltpu.VMEM((channels, chunk + 128), jnp.float32),
            pltpu.VMEM((channels, chunk + 128), jnp.float32),
            pltpu.VMEM((channels, chunk + 128), jnp.float32),
            pltpu.VMEM((channels, chunk), jnp.float32),
            pltpu.VMEM((channels, chunk), jnp.float32),
            pltpu.VMEM((channels, chunk), jnp.float32),
            pltpu.SemaphoreType.DMA,
            pltpu.SemaphoreType.DMA,
            pltpu.SemaphoreType.DMA,
            pltpu.SemaphoreType.DMA,
            pltpu.SemaphoreType.DMA,
            pltpu.SemaphoreType.DMA,
        ],
        compiler_params=pltpu.CompilerParams(use_tc_tiling_on_sc=True),
    )
    def shift_copy(wav_hbm, off_hbm, out_hbm, off_v,
                   ib0, ib1, ib2, ob0, ob1, ob2,
                   si0, si1, si2, so0, so1, so2):
        ibufs, obufs = (ib0, ib1, ib2), (ob0, ob1, ob2)
        isems, osems = (si0, si1, si2), (so0, so1, so2)
        nbuf = 3
        wid = lax.axis_index("subcore") * _NUM_CORES + lax.axis_index("core")
        pltpu.sync_copy(off_hbm, off_v)
        my_offs = off_v[wid]

        iota16 = lax.broadcasted_iota(jnp.int32, (16,), 0)
        idxs = []
        bases = []
        rem16s = []
        rots = []
        masks = []
        for k in range(pairs_per_worker):
            off = my_offs[k]
            p = wid * pairs_per_worker + k
            s = p // batch
            b = p % batch
            idxs.append((s, b))
            bases.append(pl.multiple_of((off // 128) * 128, 128))
            rem = off - bases[-1]
            r2 = jnp.bitwise_and(off, 15)
            rem16s.append(pl.multiple_of(rem - r2, 16))
            rots.append(jnp.bitwise_and(iota16 + r2, 15))
            masks.append(iota16 < 16 - r2)

        tasks = [(k, j) for k in range(pairs_per_worker)
                 for j in range(num_chunks)]
        ntasks = len(tasks)

        def in_desc(tt):
            k, j = tasks[tt]
            s, b = idxs[k]
            return pltpu.make_async_copy(
                wav_hbm.at[s, b, :, pl.ds(bases[k] + j * chunk, chunk + 128)],
                ibufs[tt % nbuf], isems[tt % nbuf])

        def out_desc(tt):
            k, j = tasks[tt]
            s, b = idxs[k]
            return pltpu.make_async_copy(
                obufs[tt % nbuf],
                out_hbm.at[s, b, :, pl.ds(j * chunk, chunk)],
                osems[tt % nbuf])

        def shift(tt):
            k, _ = tasks[tt]
            rem16, rot, mask = rem16s[k], rots[k], masks[k]
            ib, ob = ibufs[tt % nbuf], obufs[tt % nbuf]

            def body(g, carry):
                base = g * 256
                for c in range(channels):
                    pos = rem16 + base
                    vals = [ib[c, pl.ds(pos + u * 16, 16)]
                            for u in range(17)]
                    for u in range(16):
                        g0 = vals[u].at[rot].get(mode="promise_in_bounds")
                        g1 = vals[u + 1].at[rot].get(mode="promise_in_bounds")
                        ob[c, pl.ds(base + u * 16, 16)] = jnp.where(
                            mask, g0, g1)
                return carry

            lax.fori_loop(0, chunk // 256, body, 0)

        in_desc(0).start()
        in_desc(1).start()
        for tt in range(ntasks):
            if tt + 2 < ntasks:
                in_desc(tt + 2).start()
            in_desc(tt).wait()
            if tt >= nbuf:
                out_desc(tt - nbuf).wait()
            shift(tt)
            out_desc(tt).start()
        for tt in range(ntasks - nbuf, ntasks):
            out_desc(tt).wait()

    return shift_copy(wav, offs)


def kernel(wav):
    sources, batch, channels, length0 = wav.shape
    length = length0 - _SHIFT
    okey = jax.random.fold_in(jax.random.key(0), 1)
    offsets = jax.random.randint(
        okey, (sources, batch, 1, 1), 0, _SHIFT, dtype=jnp.int32
    )
    pairs = sources * batch
    num_workers = _NUM_CORES * _NUM_SUBCORES
    offs = offsets.reshape(num_workers, pairs // num_workers)
    offs = jnp.pad(offs, ((0, 0), (0, 16 - pairs // num_workers)))
    return _shift_gather(wav, offs, length)
